# Initial kernel scaffold; baseline (speedup 1.0000x reference)
import jax
import jax.numpy as jnp
from jax.experimental import pallas as pl


def kernel(edge_attr, edge_attr2, triangle_1_1_1, triangle_1_1_2, triangle_1_2_2, triangle_2_2_2, inverse_edge_1, inverse_edge_2, mlp_W1, mlp_b1, mlp_W2, mlp_b2, lin_W1, lin_b1, lin_W2, lin_b2):
    raise NotImplementedError("write your pallas kernel here")


# speedup vs baseline: 2.3419x; 2.3419x over previous
import functools

import jax
import jax.numpy as jnp
from jax.experimental import pallas as pl


_BR = 1600


def _dot(a, b):
    return jax.lax.dot(a.astype(jnp.bfloat16), b.astype(jnp.bfloat16),
                       preferred_element_type=jnp.float32)


def _mlp_body(x_ref, w1_ref, b1_ref, w2_ref, b2_ref, o_ref):
    x = x_ref[...]
    h = _dot(x, w1_ref[...]) + b1_ref[...]
    h = jnp.maximum(h, 0.0)
    o_ref[...] = _dot(h, w2_ref[...]) + b2_ref[...]


def _mlp(x, w1, b1, w2, b2):
    e, c = x.shape
    h = w1.shape[1]
    grid = (e // _BR,)
    return pl.pallas_call(
        _mlp_body,
        grid=grid,
        in_specs=[
            pl.BlockSpec((_BR, c), lambda i: (i, 0)),
            pl.BlockSpec((c, h), lambda i: (0, 0)),
            pl.BlockSpec((h,), lambda i: (0,)),
            pl.BlockSpec((h, c), lambda i: (0, 0)),
            pl.BlockSpec((c,), lambda i: (0,)),
        ],
        out_specs=pl.BlockSpec((_BR, c), lambda i: (i, 0)),
        out_shape=jax.ShapeDtypeStruct((e, c), jnp.float32),
    )(x, w1, b1, w2, b2)


def _linear_body(x_ref, w_ref, b_ref, o_ref):
    x = jnp.maximum(x_ref[...], 0.0)
    o_ref[...] = _dot(x, w_ref[...]) + b_ref[...]


def _relu_linear(x, w, b):
    e, c = x.shape
    grid = (e // _BR,)
    return pl.pallas_call(
        _linear_body,
        grid=grid,
        in_specs=[
            pl.BlockSpec((_BR, c), lambda i: (i, 0)),
            pl.BlockSpec((c, c), lambda i: (0, 0)),
            pl.BlockSpec((c,), lambda i: (0,)),
        ],
        out_specs=pl.BlockSpec((_BR, c), lambda i: (i, 0)),
        out_shape=jax.ShapeDtypeStruct((e, c), jnp.float32),
    )(x, w, b)


def _segsum(ha, ia, hb, ib, idst, num_segments):
    prod = ha[ia] * hb[ib]
    return jax.ops.segment_sum(prod, idst, num_segments=num_segments)


def kernel(edge_attr, edge_attr2, triangle_1_1_1, triangle_1_1_2, triangle_1_2_2,
           triangle_2_2_2, inverse_edge_1, inverse_edge_2,
           mlp_W1, mlp_b1, mlp_W2, mlp_b2, lin_W1, lin_b1, lin_W2, lin_b2):
    e1 = edge_attr.shape[0]
    e2 = edge_attr2.shape[0]
    ij111, ik111, kj111 = triangle_1_1_1[0], triangle_1_1_1[1], triangle_1_1_1[2]
    ij112, ik112, kj112 = triangle_1_1_2[0], triangle_1_1_2[1], triangle_1_1_2[2]
    ij122, ik122, kj122 = triangle_1_2_2[0], triangle_1_2_2[1], triangle_1_2_2[2]
    ij222, ik222, kj222 = triangle_2_2_2[0], triangle_2_2_2[1], triangle_2_2_2[2]

    def m(i, x):
        return _mlp(x, mlp_W1[0, i], mlp_b1[0, i], mlp_W2[0, i], mlp_b2[0, i])

    h0 = m(0, edge_attr)
    h1 = m(1, edge_attr)
    h2 = m(2, edge_attr2)
    h3 = m(3, edge_attr2)
    h6 = m(6, edge_attr2)
    h7 = m(7, edge_attr2)

    ms111 = _segsum(h0, ik111, h0, kj111, ij111, e1)
    ms112 = _segsum(h1, ik112, h2, kj112, ij112, e1)
    ms122 = _segsum(h3, ik122, h3, kj122, ij122, e1)
    eu = edge_attr + ms111 + ms112 + ms112[inverse_edge_1] + ms122

    h4 = m(4, eu)
    h5 = m(5, eu)

    ms211 = _segsum(h4, ij112, h4, ik112, kj112, e2)
    ms212 = _segsum(h5, ij122, h6, kj122, ik122, e2)
    ms222 = _segsum(h7, ik222, h7, kj222, ij222, e2)
    eu2 = edge_attr2 + ms211 + ms212 + ms212[inverse_edge_2] + ms222

    out1 = _relu_linear(eu, lin_W1, lin_b1)
    out2 = _relu_linear(eu2, lin_W2, lin_b2)
    return (out1, out2)
